# baseline (speedup 1.0000x reference)
import functools
import jax
import jax.numpy as jnp
from jax import lax
from jax.experimental import pallas as pl
from jax.experimental.pallas import tpu as pltpu
from jax.experimental.pallas import tpu_sc as plsc

N = 50000
D = 128
E = 300000
NPASS = 4
DC = D // NPASS
NSUB = 16
GROUP = 128
EPAD = 315392
NGRP = EPAD // (NSUB * GROUP)
KB = 22
NCHUNK = NGRP // KB
NW = KB // 2
NBUF = 3
LOOKAHEAD = 2
ROWS_PER_SUB = 3128
NROWA = ROWS_PER_SUB * NSUB
TRASH_ROW = NROWA - 1


def _sc_segment_sums(xu_flat, xi_flat, src0, dst0, src1, dst1, zeros_in):
    mesh = plsc.VectorSubcoreMesh(core_axis_name="c", subcore_axis_name="s")

    @functools.partial(
        pl.kernel,
        out_type=jax.ShapeDtypeStruct((2, NROWA, D), jnp.float32),
        mesh=mesh,
        scratch_types=[
            pltpu.VMEM((KB * GROUP,), jnp.int32),
            pltpu.VMEM((KB * GROUP,), jnp.int32),
            pltpu.VMEM((NBUF, 2 * GROUP, DC), jnp.float32),
            pltpu.VMEM_SHARED((NROWA, DC), jnp.float32),
            pltpu.SemaphoreType.DMA((NBUF,)),
            pltpu.SemaphoreType.DMA((NBUF,)),
            pltpu.SemaphoreType.DMA,
        ],
        compiler_params=pltpu.CompilerParams(use_tc_tiling_on_sc=False),
    )
    def k(xu_hbm, xi_hbm, src0_hbm, dst0_hbm, src1_hbm, dst1_hbm, z_hbm,
          agg_hbm, srcbuf, dstbuf, rows, acc, semg, sems, semz):
        c = lax.axis_index("c")
        s = lax.axis_index("s")

        def run(x_hbm, src_hbm, dst_hbm, rel):
            slab = acc.at[pl.ds(s * ROWS_PER_SUB, ROWS_PER_SUB)]
            pltpu.async_copy(z_hbm, slab, semz)
            for p in range(NPASS):
                pltpu.make_async_copy(z_hbm, slab, semz).wait()
                plsc.subcore_barrier()

                def chunk_body(ci, _):
                    pltpu.sync_copy(src_hbm.at[p, s, ci], srcbuf)
                    pltpu.sync_copy(dst_hbm.at[s, ci], dstbuf)
                    for w0 in range(LOOKAHEAD):
                        pltpu.async_copy(x_hbm.at[srcbuf.at[pl.ds(2 * GROUP * w0, 2 * GROUP)]],
                                         rows.at[w0], semg.at[w0])

                    def body(w, _):
                        b = lax.rem(w, NBUF)
                        pltpu.make_async_copy(
                            x_hbm.at[srcbuf.at[pl.ds(2 * GROUP * w, 2 * GROUP)]],
                            rows.at[b], semg.at[b]).wait()
                        pltpu.async_copy(rows.at[b],
                                         acc.at[dstbuf.at[pl.ds(2 * GROUP * w, 2 * GROUP)]],
                                         sems.at[b], add=True)

                        @pl.when(w + LOOKAHEAD < NW)
                        def _():
                            bf = lax.rem(w + LOOKAHEAD, NBUF)

                            @pl.when(w >= NBUF - LOOKAHEAD)
                            def _():
                                pltpu.make_async_copy(
                                    rows.at[bf],
                                    acc.at[dstbuf.at[pl.ds(2 * GROUP * w, 2 * GROUP)]],
                                    sems.at[bf]).wait()

                            pltpu.async_copy(
                                x_hbm.at[srcbuf.at[pl.ds(2 * GROUP * (w + LOOKAHEAD), 2 * GROUP)]],
                                rows.at[bf], semg.at[bf])
                        return 0

                    lax.fori_loop(0, NW, body, 0)
                    for b in range(NBUF):
                        pltpu.make_async_copy(
                            rows.at[b], acc.at[dstbuf.at[pl.ds(0, 2 * GROUP)]],
                            sems.at[b]).wait()
                    return 0

                lax.fori_loop(0, NCHUNK, chunk_body, 0)
                plsc.subcore_barrier()
                pltpu.sync_copy(
                    slab,
                    agg_hbm.at[rel, pl.ds(s * ROWS_PER_SUB, ROWS_PER_SUB),
                               pl.ds(p * DC, DC)],
                )
                if p < NPASS - 1:
                    pltpu.async_copy(z_hbm, slab, semz)

        @pl.when(c == 0)
        def _():
            run(xu_hbm, src0_hbm, dst0_hbm, 0)

        @pl.when(c == 1)
        def _():
            run(xi_hbm, src1_hbm, dst1_hbm, 1)

    return k(xu_flat, xi_flat, src0, dst0, src1, dst1, zeros_in)


def _tc_body(xu_ref, xi_ref, au_ref, ai_ref, wsu_ref, wsi_ref, wru_ref,
             wri_ref, bu_ref, bi_ref, ou_ref, oi_ref):
    hu = jnp.dot(xu_ref[...], wsu_ref[...], preferred_element_type=jnp.float32)
    hi = jnp.dot(xi_ref[...], wsi_ref[...], preferred_element_type=jnp.float32)
    hu += jnp.dot(au_ref[0], wru_ref[...], preferred_element_type=jnp.float32)
    hi += jnp.dot(ai_ref[0], wri_ref[...], preferred_element_type=jnp.float32)
    ou_ref[...] = jnp.maximum(hu + bu_ref[...], 0.0)
    oi_ref[...] = jnp.maximum(hi + bi_ref[...], 0.0)


def _tc_combine(x_user, x_item, agg, wsu, wsi, wru, wri, bu, bi):
    BM = 2000
    grid = (N // BM,)
    row_spec = pl.BlockSpec((BM, D), lambda m: (m, 0))
    agg_u_spec = pl.BlockSpec((1, BM, D), lambda m: (1, m, 0))
    agg_i_spec = pl.BlockSpec((1, BM, D), lambda m: (0, m, 0))
    w_spec = pl.BlockSpec((D, D), lambda m: (0, 0))
    b_spec = pl.BlockSpec((1, D), lambda m: (0, 0))
    return pl.pallas_call(
        _tc_body,
        grid=grid,
        in_specs=[row_spec, row_spec, agg_u_spec, agg_i_spec,
                  w_spec, w_spec, w_spec, w_spec, b_spec, b_spec],
        out_specs=[row_spec, row_spec],
        out_shape=[jax.ShapeDtypeStruct((N, D), jnp.float32),
                   jax.ShapeDtypeStruct((N, D), jnp.float32)],
    )(x_user, x_item, agg, agg, wsu, wsi, wru, wri, bu, bi)


def kernel(x_user, x_item, edge_index_u2i, edge_index_i2u,
           Wself_user, Wself_item, Wrel_u2i, Wrel_i2u, b_user, b_item):
    npad = EPAD - E
    poff = jnp.arange(NPASS, dtype=jnp.int32).reshape(NPASS, 1, 1, 1)

    def edge_layout(src, dst):
        src_p = jnp.concatenate(
            [src * 4, jnp.zeros((npad,), dtype=jnp.int32)]).reshape(
                NSUB, NCHUNK, KB * GROUP)
        dst_p = jnp.concatenate(
            [dst, jnp.full((npad,), TRASH_ROW, dtype=jnp.int32)]).reshape(
                NSUB, NCHUNK, KB * GROUP)
        return src_p[None] + poff, dst_p

    src0, dst0 = edge_layout(edge_index_u2i[0], edge_index_u2i[1])
    src1, dst1 = edge_layout(edge_index_i2u[0], edge_index_i2u[1])
    zeros_in = jnp.zeros((ROWS_PER_SUB, DC), dtype=jnp.float32)

    agg = _sc_segment_sums(
        x_user.reshape(N * NPASS, DC), x_item.reshape(N * NPASS, DC),
        src0, dst0, src1, dst1, zeros_in)

    out_user, out_item = _tc_combine(
        x_user, x_item, agg,
        Wself_user[-1], Wself_item[-1], Wrel_i2u[-1], Wrel_u2i[-1],
        b_user[-1].reshape(1, D), b_item[-1].reshape(1, D))
    return (out_user[None], out_item[None])


# speedup vs baseline: 2.8784x; 2.8784x over previous
import functools
import jax
import jax.numpy as jnp
from jax import lax
from jax.experimental import pallas as pl
from jax.experimental.pallas import tpu as pltpu
from jax.experimental.pallas import tpu_sc as plsc

N = 50000
D = 128
E = 300000
NPASS = 4
DC = D // NPASS
NSUB = 16
GROUP = 128
EPAD = 301056
NGRP = EPAD // (NSUB * GROUP)
KB = 21
NCHUNK = NGRP // KB
NBUF = 6
LOOKAHEAD = 4
ROWS_PER_SUB = 3128
NROWA = ROWS_PER_SUB * NSUB
TRASH_ROW = NROWA - 1


def _sc_segment_sums(xu_flat, xi_flat, src0, dst0, src1, dst1, zeros_in):
    mesh = plsc.VectorSubcoreMesh(core_axis_name="c", subcore_axis_name="s")

    @functools.partial(
        pl.kernel,
        out_type=jax.ShapeDtypeStruct((2, NROWA, D), jnp.float32),
        mesh=mesh,
        scratch_types=[
            pltpu.VMEM((KB, GROUP), jnp.int32),
            pltpu.VMEM((KB, GROUP), jnp.int32),
            pltpu.VMEM((NBUF, GROUP, DC), jnp.float32),
            pltpu.VMEM_SHARED((NROWA, DC), jnp.float32),
            pltpu.SemaphoreType.DMA((NBUF,)),
            pltpu.SemaphoreType.DMA((NBUF,)),
            pltpu.SemaphoreType.DMA,
        ],
        compiler_params=pltpu.CompilerParams(use_tc_tiling_on_sc=False),
    )
    def k(xu_hbm, xi_hbm, src0_hbm, dst0_hbm, src1_hbm, dst1_hbm, z_hbm,
          agg_hbm, srcbuf, dstbuf, rows, acc, semg, sems, semz):
        c = lax.axis_index("c")
        s = lax.axis_index("s")

        def run(x_hbm, src_hbm, dst_hbm, rel):
            slab = acc.at[pl.ds(s * ROWS_PER_SUB, ROWS_PER_SUB)]
            pltpu.async_copy(z_hbm, slab, semz)
            for p in range(NPASS):
                pltpu.make_async_copy(z_hbm, slab, semz).wait()
                plsc.subcore_barrier()

                def chunk_body(ci, _):
                    pltpu.sync_copy(src_hbm.at[p, s, ci], srcbuf)
                    pltpu.sync_copy(dst_hbm.at[s, ci], dstbuf)
                    for g0 in range(LOOKAHEAD):
                        pltpu.async_copy(x_hbm.at[srcbuf.at[g0]],
                                         rows.at[g0], semg.at[g0])

                    def body(g, _):
                        b = lax.rem(g, NBUF)
                        pltpu.make_async_copy(x_hbm.at[srcbuf.at[g]],
                                              rows.at[b], semg.at[b]).wait()
                        pltpu.async_copy(rows.at[b], acc.at[dstbuf.at[g]],
                                         sems.at[b], add=True)

                        @pl.when(g + LOOKAHEAD < KB)
                        def _():
                            bf = lax.rem(g + LOOKAHEAD, NBUF)

                            @pl.when(g >= NBUF - LOOKAHEAD)
                            def _():
                                pltpu.make_async_copy(
                                    rows.at[bf], acc.at[dstbuf.at[g]],
                                    sems.at[bf]).wait()

                            pltpu.async_copy(x_hbm.at[srcbuf.at[g + LOOKAHEAD]],
                                             rows.at[bf], semg.at[bf])
                        return 0

                    lax.fori_loop(0, KB, body, 0)
                    for b in range(NBUF):
                        pltpu.make_async_copy(rows.at[b], acc.at[dstbuf.at[0]],
                                              sems.at[b]).wait()
                    return 0

                lax.fori_loop(0, NCHUNK, chunk_body, 0)
                plsc.subcore_barrier()
                pltpu.sync_copy(
                    slab,
                    agg_hbm.at[rel, pl.ds(s * ROWS_PER_SUB, ROWS_PER_SUB),
                               pl.ds(p * DC, DC)],
                )
                if p < NPASS - 1:
                    pltpu.async_copy(z_hbm, slab, semz)

        @pl.when(c == 0)
        def _():
            run(xu_hbm, src0_hbm, dst0_hbm, 0)

        @pl.when(c == 1)
        def _():
            run(xi_hbm, src1_hbm, dst1_hbm, 1)

    return k(xu_flat, xi_flat, src0, dst0, src1, dst1, zeros_in)


def _tc_body(xu_ref, xi_ref, au_ref, ai_ref, wsu_ref, wsi_ref, wru_ref,
             wri_ref, bu_ref, bi_ref, ou_ref, oi_ref):
    hu = jnp.dot(xu_ref[...], wsu_ref[...], preferred_element_type=jnp.float32)
    hi = jnp.dot(xi_ref[...], wsi_ref[...], preferred_element_type=jnp.float32)
    hu += jnp.dot(au_ref[0], wru_ref[...], preferred_element_type=jnp.float32)
    hi += jnp.dot(ai_ref[0], wri_ref[...], preferred_element_type=jnp.float32)
    ou_ref[...] = jnp.maximum(hu + bu_ref[...], 0.0)
    oi_ref[...] = jnp.maximum(hi + bi_ref[...], 0.0)


def _tc_combine(x_user, x_item, agg, wsu, wsi, wru, wri, bu, bi):
    BM = 2000
    grid = (N // BM,)
    row_spec = pl.BlockSpec((BM, D), lambda m: (m, 0))
    agg_u_spec = pl.BlockSpec((1, BM, D), lambda m: (1, m, 0))
    agg_i_spec = pl.BlockSpec((1, BM, D), lambda m: (0, m, 0))
    w_spec = pl.BlockSpec((D, D), lambda m: (0, 0))
    b_spec = pl.BlockSpec((1, D), lambda m: (0, 0))
    return pl.pallas_call(
        _tc_body,
        grid=grid,
        in_specs=[row_spec, row_spec, agg_u_spec, agg_i_spec,
                  w_spec, w_spec, w_spec, w_spec, b_spec, b_spec],
        out_specs=[row_spec, row_spec],
        out_shape=[jax.ShapeDtypeStruct((N, D), jnp.float32),
                   jax.ShapeDtypeStruct((N, D), jnp.float32)],
    )(x_user, x_item, agg, agg, wsu, wsi, wru, wri, bu, bi)


def kernel(x_user, x_item, edge_index_u2i, edge_index_i2u,
           Wself_user, Wself_item, Wrel_u2i, Wrel_i2u, b_user, b_item):
    npad = EPAD - E
    poff = jnp.arange(NPASS, dtype=jnp.int32).reshape(NPASS, 1, 1, 1, 1)

    def edge_layout(src, dst):
        src_p = jnp.concatenate(
            [src * 4, jnp.zeros((npad,), dtype=jnp.int32)]).reshape(
                NSUB, NCHUNK, KB, GROUP)
        dst_p = jnp.concatenate(
            [dst, jnp.full((npad,), TRASH_ROW, dtype=jnp.int32)]).reshape(
                NSUB, NCHUNK, KB, GROUP)
        return src_p[None] + poff, dst_p

    src0, dst0 = edge_layout(edge_index_u2i[0], edge_index_u2i[1])
    src1, dst1 = edge_layout(edge_index_i2u[0], edge_index_i2u[1])
    zeros_in = jnp.zeros((ROWS_PER_SUB, DC), dtype=jnp.float32)

    agg = _sc_segment_sums(
        x_user.reshape(N * NPASS, DC), x_item.reshape(N * NPASS, DC),
        src0, dst0, src1, dst1, zeros_in)

    out_user, out_item = _tc_combine(
        x_user, x_item, agg,
        Wself_user[-1], Wself_item[-1], Wrel_i2u[-1], Wrel_u2i[-1],
        b_user[-1].reshape(1, D), b_item[-1].reshape(1, D))
    return (out_user[None], out_item[None])
